# Initial kernel scaffold; baseline (speedup 1.0000x reference)
import jax
import jax.numpy as jnp
from jax.experimental import pallas as pl


def kernel(context_indices, in_embed):
    raise NotImplementedError("write your pallas kernel here")


# speedup vs baseline: 1.4046x; 1.4046x over previous
import functools

import jax
import jax.numpy as jnp
from jax import lax
from jax.experimental import pallas as pl
from jax.experimental.pallas import tpu as pltpu
from jax.experimental.pallas import tpu_sc as plsc

_VOCAB = 100000
_D = 64
_B = 4096
_C = 20
_LANES = 16

_NC = 2
_NS = 16
_NW = _NC * _NS
_BPW = _B // _NW
_CB = 4
_NCHUNK = _BPW // _CB
_IDXC = _CB * _C
_NBUF = 2


def _cbow_body(idx_hbm, table_hbm, out_hbm, idx_v, bufs, out_v, sems):
    wid = lax.axis_index("s") * _NC + lax.axis_index("c")
    base = wid * _BPW
    inv_c = jnp.float32(1.0 / _C)

    pltpu.sync_copy(idx_hbm.at[wid], idx_v)

    def _gather(chunk, b):
        return pltpu.make_async_copy(
            table_hbm.at[idx_v.at[chunk]], bufs[b], sems[b])

    for b in range(_NBUF):
        _gather(b, b).start()

    @pl.loop(0, _NCHUNK, step=_NBUF, unroll=1)
    def _chunk_loop(j):
        for b in range(_NBUF):
            chunk = j + b
            _gather(chunk, b).wait()
            buf = bufs[b]
            for r in range(_CB):
                row = r * _C
                for k in range(_D // _LANES):
                    sl = pl.ds(k * _LANES, _LANES)
                    acc = buf[row, sl]
                    for c in range(1, _C):
                        acc = acc + buf[row + c, sl]
                    out_v[chunk * _CB + r, sl] = acc * inv_c

            @pl.when(chunk + _NBUF < _NCHUNK)
            def _():
                _gather(chunk + _NBUF, b).start()

    pltpu.sync_copy(out_v, out_hbm.at[pl.ds(base, _BPW)])


@jax.jit
def _cbow_sc(idx, table):
    mesh = plsc.VectorSubcoreMesh(
        core_axis_name="c", subcore_axis_name="s",
        num_cores=_NC, num_subcores=_NS)
    f = pl.kernel(
        _cbow_body,
        out_type=jax.ShapeDtypeStruct((_B, _D), jnp.float32),
        mesh=mesh,
        scratch_types=[
            pltpu.VMEM((_NCHUNK, _IDXC), jnp.int32),
            tuple(pltpu.VMEM((_IDXC, _D), jnp.float32)
                  for _ in range(_NBUF)),
            pltpu.VMEM((_BPW, _D), jnp.float32),
            tuple(pltpu.SemaphoreType.DMA for _ in range(_NBUF)),
        ],
        compiler_params=pltpu.CompilerParams(use_tc_tiling_on_sc=False),
    )
    return f(idx, table)


def kernel(context_indices, in_embed):
    idx = context_indices.astype(jnp.int32).reshape(_NW, _NCHUNK, _IDXC)
    return _cbow_sc(idx, in_embed)
